# baseline (speedup 1.0000x reference)
import jax
import jax.numpy as jnp
from jax import lax
from jax.experimental import pallas as pl
from jax.experimental.pallas import tpu as pltpu
from jax.experimental.pallas import tpu_sc as plsc

EMBED = 32
LANES = 16
NUM_CORES = 2
NUM_SUBCORES = 16
NUM_WORKERS = NUM_CORES * NUM_SUBCORES
B = 16384
L = 200
BT_ALL = B // 128
BT_W = BT_ALL // NUM_WORKERS
BW = BT_W * 128
LT = L // 8
ET = EMBED // 8
GDEPTH = 4
ODEPTH = 2


def _embedder_body(vv5, vt5, oc5, wp_hbm, wo_hbm, outp, outo,
                   vvb, vtb, ocb,
                   idx0, idx1, idx2, idx3,
                   rows0, rows1, rows2, rows3,
                   obuf0, obuf1,
                   gsem0, gsem1, gsem2, gsem3, osem0, osem1):
  wid = lax.axis_index("s") * NUM_CORES + lax.axis_index("c")
  bt0 = wid * BT_W
  iota16 = lax.iota(jnp.int32, 16)

  idx_b = (idx0, idx1, idx2, idx3)
  rows_b = (rows0, rows1, rows2, rows3)
  gsem_b = (gsem0, gsem1, gsem2, gsem3)
  obuf_b = (obuf0, obuf1)
  osem_b = (osem0, osem1)

  def prep(k):
    s = k % GDEPTH
    li = k // 2
    if k % 2 == 0:
      @plsc.parallel_loop(0, BW // LANES, unroll=4)
      def _(j):
        bt = j >> 3
        g = j & 7
        sl = pl.ds(g * 16, 16)
        idx_b[s][pl.ds(j * 16, 16)] = vvb[bt, li, sl] * vtb[bt, li, sl]
    else:
      @plsc.parallel_loop(0, BW // LANES, unroll=4)
      def _(j):
        bt = j >> 3
        g = j & 7
        idx_b[s][pl.ds(j * 16, 16)] = ocb[bt, li, pl.ds(g * 16, 16)]

  def table(k):
    return wp_hbm if k % 2 == 0 else wo_hbm

  def start_gather(k):
    s = k % GDEPTH
    pltpu.async_copy(table(k).at[idx_b[s]], rows_b[s], gsem_b[s])

  def wait_gather(k):
    s = k % GDEPTH
    pltpu.make_async_copy(table(k).at[idx_b[s]], rows_b[s], gsem_b[s]).wait()

  def transpose_relu(k):
    rows = rows_b[k % GDEPTH]
    ob = obuf_b[k % ODEPTH]

    @plsc.parallel_loop(0, BT_W * ET * 8 * 8, unroll=4)
    def _(i):
      et = i >> 8
      bt = (i >> 6) & 3
      ei = (i >> 3) & 7
      bg = i & 7
      row_idx = jnp.full((16,), bt * 128 + bg * 16, jnp.int32) + iota16
      col_idx = jnp.full((16,), et * 8 + ei, jnp.int32)
      vals = plsc.load_gather(rows, [row_idx, col_idx])
      ob[et, bt, ei, pl.ds(bg * 16, 16)] = jnp.maximum(vals, 0.0)

  def out_ref(k):
    return outp if k % 2 == 0 else outo

  def start_out(lt, k):
    s = k % ODEPTH
    l = lt * 8 + k // 2
    pltpu.async_copy(
        obuf_b[s], out_ref(k).at[l, :, pl.ds(bt0, BT_W)], osem_b[s])

  def wait_out(k):
    s = k % ODEPTH
    pltpu.make_async_copy(
        obuf_b[s], out_ref(k).at[0, :, pl.ds(bt0, BT_W)], osem_b[s]).wait()

  def lt_body(lt, carry):
    pltpu.sync_copy(vv5.at[lt, pl.ds(bt0, BT_W)], vvb)
    pltpu.sync_copy(vt5.at[lt, pl.ds(bt0, BT_W)], vtb)
    pltpu.sync_copy(oc5.at[lt, pl.ds(bt0, BT_W)], ocb)
    for k in range(GDEPTH - 1):
      prep(k)
      start_gather(k)
    for k in range(16):
      if k < 16 - (GDEPTH - 1):
        prep(k + GDEPTH - 1)
        start_gather(k + GDEPTH - 1)
      if k >= ODEPTH:
        wait_out(k)
      else:
        @pl.when(lt > 0)
        def _():
          wait_out(k)
      wait_gather(k)
      transpose_relu(k)
      start_out(lt, k)
    return carry

  lax.fori_loop(0, LT, lt_body, 0)
  wait_out(0)
  wait_out(1)


def kernel(var_val, var_type, object_class, W_pred, W_obj):
  def idx_view(a):
    return a.T.reshape(LT, 8, BT_ALL, 128).transpose(0, 2, 1, 3)

  vv5 = idx_view(var_val)
  vt5 = idx_view(var_type)
  oc5 = idx_view(object_class)

  mesh = plsc.VectorSubcoreMesh(core_axis_name="c", subcore_axis_name="s")
  run = pl.kernel(
      _embedder_body,
      out_type=(
          jax.ShapeDtypeStruct((L, ET, BT_ALL, 8, 128), jnp.float32),
          jax.ShapeDtypeStruct((L, ET, BT_ALL, 8, 128), jnp.float32),
      ),
      mesh=mesh,
      compiler_params=pltpu.CompilerParams(
          use_tc_tiling_on_sc=False, needs_layout_passes=False),
      scratch_types=[
          pltpu.VMEM((BT_W, 8, 128), jnp.int32),
          pltpu.VMEM((BT_W, 8, 128), jnp.int32),
          pltpu.VMEM((BT_W, 8, 128), jnp.int32),
          pltpu.VMEM((BW,), jnp.int32),
          pltpu.VMEM((BW,), jnp.int32),
          pltpu.VMEM((BW,), jnp.int32),
          pltpu.VMEM((BW,), jnp.int32),
          pltpu.VMEM((BW, EMBED), jnp.float32),
          pltpu.VMEM((BW, EMBED), jnp.float32),
          pltpu.VMEM((BW, EMBED), jnp.float32),
          pltpu.VMEM((BW, EMBED), jnp.float32),
          pltpu.VMEM((ET, BT_W, 8, 128), jnp.float32),
          pltpu.VMEM((ET, BT_W, 8, 128), jnp.float32),
          pltpu.SemaphoreType.DMA,
          pltpu.SemaphoreType.DMA,
          pltpu.SemaphoreType.DMA,
          pltpu.SemaphoreType.DMA,
          pltpu.SemaphoreType.DMA,
          pltpu.SemaphoreType.DMA,
      ],
  )
  o_p, o_o = run(vv5, vt5, oc5, W_pred, W_obj)

  def out_view(z):
    return z.transpose(2, 4, 0, 1, 3).reshape(B, L, EMBED)

  return (out_view(o_p), out_view(o_o))


# speedup vs baseline: 2.0994x; 2.0994x over previous
import functools

import jax
import jax.numpy as jnp
from jax import lax
from jax.experimental import pallas as pl
from jax.experimental.pallas import tpu as pltpu
from jax.experimental.pallas import tpu_sc as plsc

EMBED = 32
LANES = 16
NUM_CORES = 2
NUM_SUBCORES = 16
NUM_WORKERS = NUM_CORES * NUM_SUBCORES
CHUNK = 1024
NBUF = 2


def _relu_slot(rows_v, b):
  @plsc.parallel_loop(0, CHUNK, unroll=8)
  def _(i):
    for h in range(EMBED // LANES):
      sl = (b, i, pl.ds(h * LANES, LANES))
      rows_v[sl] = jnp.maximum(rows_v[sl], 0.0)


def _phase(nchunks, base, prep, table, out, idx_v, rows_v, gsems, osems):
  def start_gather(slot):
    pltpu.async_copy(table.at[idx_v.at[slot]], rows_v.at[slot], gsems[slot])

  def wait_gather(slot):
    pltpu.make_async_copy(
        table.at[idx_v.at[slot]], rows_v.at[slot], gsems[slot]).wait()

  def start_out(c, slot):
    off = base + c * CHUNK
    pltpu.async_copy(rows_v.at[slot],
                     out.at[pl.ds(off, CHUNK), pl.ds(0, EMBED)], osems[slot])

  def wait_out(slot):
    pltpu.make_async_copy(
        rows_v.at[slot],
        out.at[pl.ds(base, CHUNK), pl.ds(0, EMBED)], osems[slot]).wait()

  prep(0, 0)
  start_gather(0)

  def outer(g, carry):
    for b in range(NBUF):
      c = g * NBUF + b
      cn = c + 1
      sn = (b + 1) % NBUF

      @pl.when(cn < nchunks)
      def _():
        @pl.when(cn >= NBUF)
        def _():
          wait_out(sn)
        prep(cn, sn)
        start_gather(sn)

      wait_gather(b)
      _relu_slot(rows_v, b)
      start_out(c, b)
    return carry

  lax.fori_loop(0, nchunks // NBUF, outer, 0)
  for b in range(NBUF):
    wait_out(b)


def _pred_body(vv_hbm, vt_hbm, wp_hbm, outp_hbm,
               idx_v, vv_v, vt_v, rows_v,
               gsem0, gsem1, osem0, osem1, *, n_rows):
  rows_per_w = n_rows // NUM_WORKERS
  nchunks = rows_per_w // CHUNK
  wid = lax.axis_index("s") * NUM_CORES + lax.axis_index("c")
  base = wid * rows_per_w

  def prep_pred(c, slot):
    off = base + c * CHUNK
    pltpu.sync_copy(vv_hbm.at[pl.ds(off, CHUNK)], vv_v)
    pltpu.sync_copy(vt_hbm.at[pl.ds(off, CHUNK)], vt_v)

    @plsc.parallel_loop(0, CHUNK // LANES, unroll=8)
    def _(j):
      sl = pl.ds(j * LANES, LANES)
      idx_v[slot, sl] = vv_v[sl] * vt_v[sl]

  _phase(nchunks, base, prep_pred, wp_hbm, outp_hbm,
         idx_v, rows_v, (gsem0, gsem1), (osem0, osem1))


def _obj_body(oc_hbm, wo_hbm, outo_hbm,
              idx_v, rows_v,
              gsem0, gsem1, osem0, osem1, *, n_rows):
  rows_per_w = n_rows // NUM_WORKERS
  nchunks = rows_per_w // CHUNK
  wid = lax.axis_index("s") * NUM_CORES + lax.axis_index("c")
  base = wid * rows_per_w

  def prep_obj(c, slot):
    off = base + c * CHUNK
    pltpu.sync_copy(oc_hbm.at[pl.ds(off, CHUNK)], idx_v.at[slot])

  _phase(nchunks, base, prep_obj, wo_hbm, outo_hbm,
         idx_v, rows_v, (gsem0, gsem1), (osem0, osem1))


def kernel(var_val, var_type, object_class, W_pred, W_obj):
  B, L = var_val.shape
  n = B * L
  vv = var_val.reshape(n)
  vt = var_type.reshape(n)
  oc = object_class.reshape(n)

  mesh = plsc.VectorSubcoreMesh(core_axis_name="c", subcore_axis_name="s")
  out_t = jax.ShapeDtypeStruct((n, 128), jnp.float32)
  run_pred = pl.kernel(
      functools.partial(_pred_body, n_rows=n),
      out_type=out_t,
      mesh=mesh,
      compiler_params=pltpu.CompilerParams(use_tc_tiling_on_sc=False),
      scratch_types=[
          pltpu.VMEM((NBUF, CHUNK), jnp.int32),
          pltpu.VMEM((CHUNK,), jnp.int32),
          pltpu.VMEM((CHUNK,), jnp.int32),
          pltpu.VMEM((NBUF, CHUNK, EMBED), jnp.float32),
          pltpu.SemaphoreType.DMA,
          pltpu.SemaphoreType.DMA,
          pltpu.SemaphoreType.DMA,
          pltpu.SemaphoreType.DMA,
      ],
  )
  run_obj = pl.kernel(
      functools.partial(_obj_body, n_rows=n),
      out_type=out_t,
      mesh=mesh,
      compiler_params=pltpu.CompilerParams(use_tc_tiling_on_sc=False),
      scratch_types=[
          pltpu.VMEM((NBUF, CHUNK), jnp.int32),
          pltpu.VMEM((NBUF, CHUNK, EMBED), jnp.float32),
          pltpu.SemaphoreType.DMA,
          pltpu.SemaphoreType.DMA,
          pltpu.SemaphoreType.DMA,
          pltpu.SemaphoreType.DMA,
      ],
  )
  h_p = run_pred(vv, vt, W_pred)
  h_o = run_obj(oc, W_obj)
  h_p = h_p[:, :EMBED].reshape(B, L, EMBED)
  h_o = h_o[:, :EMBED].reshape(B, L, EMBED)
  return (h_p, h_o)


# speedup vs baseline: 2.1169x; 1.0084x over previous
import functools

import jax
import jax.numpy as jnp
from jax import lax
from jax.experimental import pallas as pl
from jax.experimental.pallas import tpu as pltpu
from jax.experimental.pallas import tpu_sc as plsc

EMBED = 32
LANES = 16
NUM_CORES = 2
NUM_SUBCORES = 16
NUM_WORKERS = NUM_CORES * NUM_SUBCORES
CHUNK = 1280
NBUF = 2


def _relu_slot(rows_v, b):
  @plsc.parallel_loop(0, CHUNK, unroll=8)
  def _(i):
    for h in range(EMBED // LANES):
      sl = (b, i, pl.ds(h * LANES, LANES))
      rows_v[sl] = jnp.maximum(rows_v[sl], 0.0)


def _phase(nchunks, base, prep, table, out, idx_v, rows_v, gsems, osems):
  def start_gather(slot):
    pltpu.async_copy(table.at[idx_v.at[slot]], rows_v.at[slot], gsems[slot])

  def wait_gather(slot):
    pltpu.make_async_copy(
        table.at[idx_v.at[slot]], rows_v.at[slot], gsems[slot]).wait()

  def start_out(c, slot):
    off = base + c * CHUNK
    pltpu.async_copy(rows_v.at[slot],
                     out.at[pl.ds(off, CHUNK), pl.ds(0, EMBED)], osems[slot])

  def wait_out(slot):
    pltpu.make_async_copy(
        rows_v.at[slot],
        out.at[pl.ds(base, CHUNK), pl.ds(0, EMBED)], osems[slot]).wait()

  prep(0, 0)
  start_gather(0)

  def outer(g, carry):
    for b in range(NBUF):
      c = g * NBUF + b
      cn = c + 1
      sn = (b + 1) % NBUF

      @pl.when(cn < nchunks)
      def _():
        @pl.when(cn >= NBUF)
        def _():
          wait_out(sn)
        prep(cn, sn)
        start_gather(sn)

      wait_gather(b)
      _relu_slot(rows_v, b)
      start_out(c, b)
    return carry

  lax.fori_loop(0, nchunks // NBUF, outer, 0)
  for b in range(NBUF):
    wait_out(b)


def _pred_body(vv_hbm, vt_hbm, wp_hbm, outp_hbm,
               idx_v, vv_v, vt_v, rows_v,
               gsem0, gsem1, osem0, osem1, *, n_rows):
  rows_per_w = n_rows // NUM_WORKERS
  nchunks = rows_per_w // CHUNK
  wid = lax.axis_index("s") * NUM_CORES + lax.axis_index("c")
  base = wid * rows_per_w

  def prep_pred(c, slot):
    off = base + c * CHUNK
    pltpu.sync_copy(vv_hbm.at[pl.ds(off, CHUNK)], vv_v)
    pltpu.sync_copy(vt_hbm.at[pl.ds(off, CHUNK)], vt_v)

    @plsc.parallel_loop(0, CHUNK // LANES, unroll=8)
    def _(j):
      sl = pl.ds(j * LANES, LANES)
      idx_v[slot, sl] = vv_v[sl] * vt_v[sl]

  _phase(nchunks, base, prep_pred, wp_hbm, outp_hbm,
         idx_v, rows_v, (gsem0, gsem1), (osem0, osem1))


def _obj_body(oc_hbm, wo_hbm, outo_hbm,
              idx_v, rows_v,
              gsem0, gsem1, osem0, osem1, *, n_rows):
  rows_per_w = n_rows // NUM_WORKERS
  nchunks = rows_per_w // CHUNK
  wid = lax.axis_index("s") * NUM_CORES + lax.axis_index("c")
  base = wid * rows_per_w

  def prep_obj(c, slot):
    off = base + c * CHUNK
    pltpu.sync_copy(oc_hbm.at[pl.ds(off, CHUNK)], idx_v.at[slot])

  _phase(nchunks, base, prep_obj, wo_hbm, outo_hbm,
         idx_v, rows_v, (gsem0, gsem1), (osem0, osem1))


def kernel(var_val, var_type, object_class, W_pred, W_obj):
  B, L = var_val.shape
  n = B * L
  vv = var_val.reshape(n)
  vt = var_type.reshape(n)
  oc = object_class.reshape(n)

  mesh = plsc.VectorSubcoreMesh(core_axis_name="c", subcore_axis_name="s")
  out_t = jax.ShapeDtypeStruct((n, 128), jnp.float32)
  run_pred = pl.kernel(
      functools.partial(_pred_body, n_rows=n),
      out_type=out_t,
      mesh=mesh,
      compiler_params=pltpu.CompilerParams(use_tc_tiling_on_sc=False),
      scratch_types=[
          pltpu.VMEM((NBUF, CHUNK), jnp.int32),
          pltpu.VMEM((CHUNK,), jnp.int32),
          pltpu.VMEM((CHUNK,), jnp.int32),
          pltpu.VMEM((NBUF, CHUNK, EMBED), jnp.float32),
          pltpu.SemaphoreType.DMA,
          pltpu.SemaphoreType.DMA,
          pltpu.SemaphoreType.DMA,
          pltpu.SemaphoreType.DMA,
      ],
  )
  run_obj = pl.kernel(
      functools.partial(_obj_body, n_rows=n),
      out_type=out_t,
      mesh=mesh,
      compiler_params=pltpu.CompilerParams(use_tc_tiling_on_sc=False),
      scratch_types=[
          pltpu.VMEM((NBUF, CHUNK), jnp.int32),
          pltpu.VMEM((NBUF, CHUNK, EMBED), jnp.float32),
          pltpu.SemaphoreType.DMA,
          pltpu.SemaphoreType.DMA,
          pltpu.SemaphoreType.DMA,
          pltpu.SemaphoreType.DMA,
      ],
  )
  h_p = run_pred(vv, vt, W_pred)
  h_o = run_obj(oc, W_obj)
  h_p = h_p[:, :EMBED].reshape(B, L, EMBED)
  h_o = h_o[:, :EMBED].reshape(B, L, EMBED)
  return (h_p, h_o)
